# baseline (speedup 1.0000x reference)
import functools

import jax
import jax.numpy as jnp
from jax import lax
from jax.experimental import pallas as pl
from jax.experimental.pallas import tpu as pltpu
from jax.experimental.pallas import tpu_sc as plsc

VOCAB_ = 1000000
D = 64
B_ = 4096
S_ = 200

NW = 32
CHUNK = 100
N_ROWS = B_ * S_
N_CHUNKS = N_ROWS // CHUNK
CPW = N_CHUNKS // NW
NB = 4
LANES = 16


def _pos_encoding(seq_len, d_model):
    pos = jnp.arange(0, seq_len, dtype=jnp.float32)[:, None]
    dim = jnp.arange(0, d_model, dtype=jnp.float32)
    result = jnp.zeros((seq_len, d_model), dtype=jnp.float32)
    even = jnp.sin(pos / 10000 ** (dim[0::2] / d_model))
    odd = jnp.cos(pos / 10000 ** (dim[1::2] / d_model))
    result = result.at[:, 0::2].set(even)
    result = result.at[:, 1::2].set(odd)
    return result


@jax.jit
def kernel(x, table):
    idx = x.reshape(NW, CPW, CHUNK).astype(jnp.int32)
    pe = _pos_encoding(S_, D).reshape(2, CHUNK, D)

    mesh = plsc.VectorSubcoreMesh(core_axis_name="c", subcore_axis_name="s")

    scratch = [
        pltpu.VMEM((CPW, CHUNK), jnp.int32),
        pltpu.VMEM((2, CHUNK, D), jnp.float32),
    ]
    for _ in range(NB):
        scratch.append(pltpu.VMEM((CHUNK, D), jnp.float32))
    for _ in range(2 * NB):
        scratch.append(pltpu.SemaphoreType.DMA)

    @functools.partial(
        pl.kernel,
        out_type=jax.ShapeDtypeStruct((N_CHUNKS, CHUNK, D), jnp.float32),
        mesh=mesh,
        scratch_types=scratch,
        compiler_params=pltpu.CompilerParams(use_tc_tiling_on_sc=False),
    )
    def run(table_hbm, idx_hbm, pe_hbm, out_hbm, idx_v, pe_v, *rest):
        bufs = rest[:NB]
        gsem = rest[NB:2 * NB]
        ssem = rest[2 * NB:]

        wid = lax.axis_index("c") * 16 + lax.axis_index("s")
        base = wid * CPW

        pltpu.sync_copy(idx_hbm.at[wid], idx_v)
        pltpu.sync_copy(pe_hbm, pe_v)

        def issue_gather(g, b):
            pltpu.async_copy(table_hbm.at[idx_v.at[g]], bufs[b], gsem[b])

        def wait_gather(g, b):
            pltpu.make_async_copy(table_hbm.at[idx_v.at[g]], bufs[b],
                                  gsem[b]).wait()

        def issue_store(g, b):
            pltpu.async_copy(bufs[b], out_hbm.at[base + g], ssem[b])

        def wait_store(b):
            pltpu.make_async_copy(bufs[b], out_hbm.at[0], ssem[b]).wait()

        for b in range(NB - 1):
            issue_gather(b, b)

        @pl.loop(0, CPW, step=NB)
        def chunk_group(g0):
            for b in range(NB):
                g = g0 + b
                wait_gather(g, b)
                phase = b % 2
                pe_b = pe_v.at[phase]

                @pl.loop(0, CHUNK)
                def add_pe(r):
                    for c in range(D // LANES):
                        slc = pl.ds(c * LANES, LANES)
                        plsc.addupdate(bufs[b].at[r, slc], pe_b[r, slc])

                issue_store(g, b)

                f = g + (NB - 1)
                bf = (b + NB - 1) % NB

                @pl.when(f < CPW)
                def _():
                    @pl.when(g >= 1)
                    def _():
                        wait_store(bf)

                    issue_gather(f, bf)

        for b in range(NB):
            wait_store(b)

    out = run(table, idx, pe)
    return out.reshape(B_, S_, D)
